# Initial kernel scaffold; baseline (speedup 1.0000x reference)
import jax
import jax.numpy as jnp
from jax.experimental import pallas as pl


def kernel(msgs_0, msgs_1, index, num_nodes, W1, b1, W2, b2, L1W, S1, sb1, S2, sb2):
    raise NotImplementedError("write your pallas kernel here")


# speedup vs baseline: 18.6736x; 18.6736x over previous
import functools

import jax
import jax.numpy as jnp
from jax import lax
from jax.experimental import pallas as pl
from jax.experimental.pallas import tpu as pltpu
from jax.experimental.pallas import tpu_sc as plsc

F = 128
E = 320000
N = 10000
NPAD = 10240
NC, NS = 2, 16
EPT = E // NC // NS
B = 128
NFULL = EPT // B
TAIL = EPT - NFULL * B
ROWS_PT = NPAD // NS
INV_SQRT_DEG = 1.0 / (32.0 ** 0.5)

_MESH = dict(core_axis_name="c", subcore_axis_name="s",
             num_cores=NC, num_subcores=NS)


def _zero_fill(zbuf):
  def zrow(r, carry):
    for kk in range(F // 16):
      if len(zbuf.shape) == 3:
        zbuf[r, 0, pl.ds(kk * 16, 16)] = jnp.zeros((16,), jnp.float32)
      else:
        zbuf[r, pl.ds(kk * 16, 16)] = jnp.zeros((16,), jnp.float32)
    return carry
  lax.fori_loop(0, B, zrow, 0)


def _sc_sum_m0(m0, idx):
  mesh = plsc.VectorSubcoreMesh(**_MESH)

  @functools.partial(
      pl.kernel,
      out_type=jax.ShapeDtypeStruct((NC, NPAD, F), jnp.float32),
      mesh=mesh,
      scratch_types=[
          pltpu.VMEM_SHARED((NPAD, F), jnp.float32),
          pltpu.VMEM((B, F), jnp.float32),
          pltpu.VMEM((TAIL, F), jnp.float32),
          pltpu.VMEM((B,), jnp.int32),
          pltpu.VMEM((TAIL,), jnp.int32),
          pltpu.VMEM((B, F), jnp.float32),
      ],
  )
  def k(m0_hbm, idx_hbm, out_hbm, acc, rows, rowst, idxb, idxt, zbuf):
    s = lax.axis_index("s")
    c = lax.axis_index("c")
    _zero_fill(zbuf)
    for z in range(ROWS_PT // B):
      pltpu.sync_copy(zbuf, acc.at[pl.ds(s * ROWS_PT + z * B, B)])
    plsc.subcore_barrier()

    base = c * (E // NC) + s * EPT

    def body(b, carry):
      e0 = base + b * B
      pltpu.sync_copy(idx_hbm.at[pl.ds(e0, B)], idxb)
      pltpu.sync_copy(m0_hbm.at[pl.ds(e0, B)], rows)
      pltpu.sync_copy(rows, acc.at[idxb], add=True)
      return carry
    lax.fori_loop(0, NFULL, body, 0)

    et = base + NFULL * B
    pltpu.sync_copy(idx_hbm.at[pl.ds(et, TAIL)], idxt)
    pltpu.sync_copy(m0_hbm.at[pl.ds(et, TAIL)], rowst)
    pltpu.sync_copy(rowst, acc.at[idxt], add=True)

    plsc.subcore_barrier()
    pltpu.sync_copy(acc.at[pl.ds(s * ROWS_PT, ROWS_PT)],
                    out_hbm.at[c, pl.ds(s * ROWS_PT, ROWS_PT)])

  return k(m0, idx)


def _sc_sum_m1(m1, idx):
  mesh = plsc.VectorSubcoreMesh(**_MESH)

  @functools.partial(
      pl.kernel,
      out_type=jax.ShapeDtypeStruct((3, NC, NPAD, 1, F), jnp.float32),
      mesh=mesh,
      scratch_types=[
          pltpu.VMEM_SHARED((NPAD, 1, F), jnp.float32),
          pltpu.VMEM((B, 1, F), jnp.float32),
          pltpu.VMEM((TAIL, 1, F), jnp.float32),
          pltpu.VMEM((B,), jnp.int32),
          pltpu.VMEM((TAIL,), jnp.int32),
          pltpu.VMEM((B, 1, F), jnp.float32),
      ],
  )
  def k(m1_hbm, idx_hbm, out_hbm, acc, rows, rowst, idxb, idxt, zbuf):
    s = lax.axis_index("s")
    c = lax.axis_index("c")
    _zero_fill(zbuf)
    base = c * (E // NC) + s * EPT

    for j in range(3):
      for z in range(ROWS_PT // B):
        pltpu.sync_copy(zbuf, acc.at[pl.ds(s * ROWS_PT + z * B, B)])
      plsc.subcore_barrier()

      def body(b, carry):
        e0 = base + b * B
        pltpu.sync_copy(idx_hbm.at[pl.ds(e0, B)], idxb)
        pltpu.sync_copy(m1_hbm.at[pl.ds(e0, B), pl.ds(j, 1)], rows)
        pltpu.sync_copy(rows, acc.at[idxb], add=True)
        return carry
      lax.fori_loop(0, NFULL, body, 0)

      et = base + NFULL * B
      pltpu.sync_copy(idx_hbm.at[pl.ds(et, TAIL)], idxt)
      pltpu.sync_copy(m1_hbm.at[pl.ds(et, TAIL), pl.ds(j, 1)], rowst)
      pltpu.sync_copy(rowst, acc.at[idxt], add=True)

      plsc.subcore_barrier()
      pltpu.sync_copy(acc.at[pl.ds(s * ROWS_PT, ROWS_PT)],
                      out_hbm.at[j, c, pl.ds(s * ROWS_PT, ROWS_PT)])
      plsc.subcore_barrier()

  return k(m1, idx)


BLK = 1000


def _tc_body(da, db, w1a, w1b, b1r, w2, b2r, lw, s1, sb1r, s2, sb2r,
             dh0, dh1):
  cs = INV_SQRT_DEG
  xa = da[...]
  xb = db[...]
  den0 = (xa[0] + xa[1]) * cs
  a = (xb[0, 0, :, 0] + xb[0, 1, :, 0]) * cs
  b = (xb[1, 0, :, 0] + xb[1, 1, :, 0]) * cs
  d = (xb[2, 0, :, 0] + xb[2, 1, :, 0]) * cs
  inv1 = jnp.sqrt(a * a + b * b + d * d + 1e-8)
  f32 = jnp.float32
  h = (jnp.dot(den0, w1a[...], preferred_element_type=f32)
       + jnp.dot(inv1, w1b[...], preferred_element_type=f32) + b1r[...])
  h = h * jax.nn.sigmoid(h)
  dh0v = jnp.dot(h, w2[...], preferred_element_type=f32) + b2r[...]
  sh = jnp.dot(dh0v, s1[...], preferred_element_type=f32) + sb1r[...]
  sh = sh * jax.nn.sigmoid(sh)
  alpha = jnp.dot(sh, s2[...], preferred_element_type=f32) + sb2r[...]
  dh0[...] = dh0v
  dh1[...] = jnp.stack(
      [jnp.dot(a, lw[...], preferred_element_type=f32) * alpha,
       jnp.dot(b, lw[...], preferred_element_type=f32) * alpha,
       jnp.dot(d, lw[...], preferred_element_type=f32) * alpha], axis=1)


def _tc_mlp(den_a, den_b, w1a, w1b, b1, w2, b2, lw, s1, sb1, s2, sb2):
  wspec = lambda shape: pl.BlockSpec(shape, lambda i: (0,) * len(shape))
  return pl.pallas_call(
      _tc_body,
      grid=(N // BLK,),
      in_specs=[
          pl.BlockSpec((NC, BLK, F), lambda i: (0, i, 0)),
          pl.BlockSpec((3, NC, BLK, 1, F), lambda i: (0, 0, i, 0, 0)),
          wspec((F, F)), wspec((F, F)), wspec((1, F)),
          wspec((F, F)), wspec((1, F)), wspec((F, F)),
          wspec((F, F)), wspec((1, F)), wspec((F, F)), wspec((1, F)),
      ],
      out_specs=[
          pl.BlockSpec((BLK, F), lambda i: (i, 0)),
          pl.BlockSpec((BLK, 3, F), lambda i: (i, 0, 0)),
      ],
      out_shape=[
          jax.ShapeDtypeStruct((N, F), jnp.float32),
          jax.ShapeDtypeStruct((N, 3, F), jnp.float32),
      ],
      compiler_params=pltpu.CompilerParams(
          dimension_semantics=("arbitrary",)),
  )(den_a, den_b, w1a, w1b, b1, w2, b2, lw, s1, sb1, s2, sb2)


def kernel(msgs_0, msgs_1, index, num_nodes, W1, b1, W2, b2, L1W, S1, sb1, S2,
           sb2):
  idxc = jnp.minimum(index, num_nodes - 1).astype(jnp.int32)
  den_a = _sc_sum_m0(msgs_0, idxc)
  den_b = _sc_sum_m1(msgs_1, idxc)
  w1t = W1.T
  dh0, dh1 = _tc_mlp(
      den_a, den_b,
      w1t[:F], w1t[F:], b1.reshape(1, F),
      W2.T, b2.reshape(1, F), L1W.T,
      S1.T, sb1.reshape(1, F), S2.T, sb2.reshape(1, F))
  return (dh0, dh1)


# speedup vs baseline: 21.6394x; 1.1588x over previous
import functools

import jax
import jax.numpy as jnp
from jax import lax
from jax.experimental import pallas as pl
from jax.experimental.pallas import tpu as pltpu
from jax.experimental.pallas import tpu_sc as plsc

F = 128
E = 320000
N = 10000
NPAD = 10240
NC, NS = 2, 16
EPT = E // NC // NS
B = 128
NFULL = EPT // B
TAIL = EPT - NFULL * B
ROWS_PT = NPAD // NS
ZB = 64
INV_SQRT_DEG = 1.0 / (32.0 ** 0.5)

_MESH = dict(core_axis_name="c", subcore_axis_name="s",
             num_cores=NC, num_subcores=NS)


def _zero_fill(zbuf):
  def zrow(r, carry):
    for kk in range(F // 16):
      if len(zbuf.shape) == 3:
        zbuf[r, 0, pl.ds(kk * 16, 16)] = jnp.zeros((16,), jnp.float32)
      else:
        zbuf[r, pl.ds(kk * 16, 16)] = jnp.zeros((16,), jnp.float32)
    return carry
  lax.fori_loop(0, ZB, zrow, 0)


def _chunk_pipeline(idx_hbm, acc, base, rows, idxs, sls, sss,
                    src_at, rowst, idxt):
  def load(b, p):
    e0 = base + b * B
    pltpu.async_copy(idx_hbm.at[pl.ds(e0, B)], idxs[p], sls[p])
    pltpu.async_copy(src_at(e0, B), rows[p], sls[p])

  def wait_load(p):
    pltpu.make_async_copy(idx_hbm.at[pl.ds(base, B)], idxs[p], sls[p]).wait()
    pltpu.make_async_copy(src_at(base, B), rows[p], sls[p]).wait()

  def scat(p):
    pltpu.async_copy(rows[p], acc.at[idxs[p]], sss[p], add=True)

  def wait_scat(p):
    pltpu.make_async_copy(rows[p], acc.at[idxs[p]], sss[p]).wait()

  load(0, 0)
  load(1, 1)
  wait_load(0)
  scat(0)
  wait_load(1)
  scat(1)

  def body(g, carry):
    x = 2 * g
    wait_scat(0)
    load(x, 0)
    wait_scat(1)
    load(x + 1, 1)
    wait_load(0)
    scat(0)
    wait_load(1)
    scat(1)
    return carry
  lax.fori_loop(1, NFULL // 2, body, 0)

  wait_scat(0)
  wait_scat(1)

  et = base + NFULL * B
  pltpu.sync_copy(idx_hbm.at[pl.ds(et, TAIL)], idxt)
  pltpu.sync_copy(src_at(et, TAIL), rowst)
  pltpu.sync_copy(rowst, acc.at[idxt], add=True)


def _sc_sum_m0(m0, idx):
  mesh = plsc.VectorSubcoreMesh(**_MESH)

  @functools.partial(
      pl.kernel,
      out_type=jax.ShapeDtypeStruct((NC, NPAD, F), jnp.float32),
      mesh=mesh,
      scratch_types=[
          pltpu.VMEM_SHARED((NPAD, F), jnp.float32),
          pltpu.VMEM((B, F), jnp.float32),
          pltpu.VMEM((B, F), jnp.float32),
          pltpu.VMEM((TAIL, F), jnp.float32),
          pltpu.VMEM((B,), jnp.int32),
          pltpu.VMEM((B,), jnp.int32),
          pltpu.VMEM((TAIL,), jnp.int32),
          pltpu.VMEM((ZB, F), jnp.float32),
          pltpu.SemaphoreType.DMA,
          pltpu.SemaphoreType.DMA,
          pltpu.SemaphoreType.DMA,
          pltpu.SemaphoreType.DMA,
      ],
  )
  def k(m0_hbm, idx_hbm, out_hbm, acc, rows0, rows1, rowst,
        idx0, idx1, idxt, zbuf, sl0, sl1, ss0, ss1):
    s = lax.axis_index("s")
    c = lax.axis_index("c")
    _zero_fill(zbuf)
    for z in range(ROWS_PT // ZB):
      pltpu.sync_copy(zbuf, acc.at[pl.ds(s * ROWS_PT + z * ZB, ZB)])
    plsc.subcore_barrier()

    base = c * (E // NC) + s * EPT
    _chunk_pipeline(idx_hbm, acc, base, (rows0, rows1), (idx0, idx1),
                    (sl0, sl1), (ss0, ss1),
                    lambda e0, n: m0_hbm.at[pl.ds(e0, n)], rowst, idxt)

    plsc.subcore_barrier()
    pltpu.sync_copy(acc.at[pl.ds(s * ROWS_PT, ROWS_PT)],
                    out_hbm.at[c, pl.ds(s * ROWS_PT, ROWS_PT)])

  return k(m0, idx)


def _sc_sum_m1(m1, idx):
  mesh = plsc.VectorSubcoreMesh(**_MESH)

  @functools.partial(
      pl.kernel,
      out_type=jax.ShapeDtypeStruct((3, NC, NPAD, 1, F), jnp.float32),
      mesh=mesh,
      scratch_types=[
          pltpu.VMEM_SHARED((NPAD, 1, F), jnp.float32),
          pltpu.VMEM((B, 1, F), jnp.float32),
          pltpu.VMEM((B, 1, F), jnp.float32),
          pltpu.VMEM((TAIL, 1, F), jnp.float32),
          pltpu.VMEM((B,), jnp.int32),
          pltpu.VMEM((B,), jnp.int32),
          pltpu.VMEM((TAIL,), jnp.int32),
          pltpu.VMEM((ZB, 1, F), jnp.float32),
          pltpu.SemaphoreType.DMA,
          pltpu.SemaphoreType.DMA,
          pltpu.SemaphoreType.DMA,
          pltpu.SemaphoreType.DMA,
      ],
  )
  def k(m1_hbm, idx_hbm, out_hbm, acc, rows0, rows1, rowst,
        idx0, idx1, idxt, zbuf, sl0, sl1, ss0, ss1):
    s = lax.axis_index("s")
    c = lax.axis_index("c")
    _zero_fill(zbuf)
    base = c * (E // NC) + s * EPT

    for j in range(3):
      for z in range(ROWS_PT // ZB):
        pltpu.sync_copy(zbuf, acc.at[pl.ds(s * ROWS_PT + z * ZB, ZB)])
      plsc.subcore_barrier()

      _chunk_pipeline(
          idx_hbm, acc, base, (rows0, rows1), (idx0, idx1),
          (sl0, sl1), (ss0, ss1),
          lambda e0, n: m1_hbm.at[pl.ds(e0, n), pl.ds(j, 1)], rowst, idxt)

      plsc.subcore_barrier()
      pltpu.sync_copy(acc.at[pl.ds(s * ROWS_PT, ROWS_PT)],
                      out_hbm.at[j, c, pl.ds(s * ROWS_PT, ROWS_PT)])
      plsc.subcore_barrier()

  return k(m1, idx)


BLK = 1000


def _tc_body(da, db, w1a, w1b, b1r, w2, b2r, lw, s1, sb1r, s2, sb2r,
             dh0, dh1):
  cs = INV_SQRT_DEG
  xa = da[...]
  xb = db[...]
  den0 = (xa[0] + xa[1]) * cs
  a = (xb[0, 0, :, 0] + xb[0, 1, :, 0]) * cs
  b = (xb[1, 0, :, 0] + xb[1, 1, :, 0]) * cs
  d = (xb[2, 0, :, 0] + xb[2, 1, :, 0]) * cs
  inv1 = jnp.sqrt(a * a + b * b + d * d + 1e-8)
  f32 = jnp.float32
  h = (jnp.dot(den0, w1a[...], preferred_element_type=f32)
       + jnp.dot(inv1, w1b[...], preferred_element_type=f32) + b1r[...])
  h = h * jax.nn.sigmoid(h)
  dh0v = jnp.dot(h, w2[...], preferred_element_type=f32) + b2r[...]
  sh = jnp.dot(dh0v, s1[...], preferred_element_type=f32) + sb1r[...]
  sh = sh * jax.nn.sigmoid(sh)
  alpha = jnp.dot(sh, s2[...], preferred_element_type=f32) + sb2r[...]
  dh0[...] = dh0v
  dh1[...] = jnp.stack(
      [jnp.dot(a, lw[...], preferred_element_type=f32) * alpha,
       jnp.dot(b, lw[...], preferred_element_type=f32) * alpha,
       jnp.dot(d, lw[...], preferred_element_type=f32) * alpha], axis=1)


def _tc_mlp(den_a, den_b, w1a, w1b, b1, w2, b2, lw, s1, sb1, s2, sb2):
  wspec = lambda shape: pl.BlockSpec(shape, lambda i: (0,) * len(shape))
  return pl.pallas_call(
      _tc_body,
      grid=(N // BLK,),
      in_specs=[
          pl.BlockSpec((NC, BLK, F), lambda i: (0, i, 0)),
          pl.BlockSpec((3, NC, BLK, 1, F), lambda i: (0, 0, i, 0, 0)),
          wspec((F, F)), wspec((F, F)), wspec((1, F)),
          wspec((F, F)), wspec((1, F)), wspec((F, F)),
          wspec((F, F)), wspec((1, F)), wspec((F, F)), wspec((1, F)),
      ],
      out_specs=[
          pl.BlockSpec((BLK, F), lambda i: (i, 0)),
          pl.BlockSpec((BLK, 3, F), lambda i: (i, 0, 0)),
      ],
      out_shape=[
          jax.ShapeDtypeStruct((N, F), jnp.float32),
          jax.ShapeDtypeStruct((N, 3, F), jnp.float32),
      ],
      compiler_params=pltpu.CompilerParams(
          dimension_semantics=("arbitrary",)),
  )(den_a, den_b, w1a, w1b, b1, w2, b2, lw, s1, sb1, s2, sb2)


def kernel(msgs_0, msgs_1, index, num_nodes, W1, b1, W2, b2, L1W, S1, sb1, S2,
           sb2):
  idxc = jnp.minimum(index, num_nodes - 1).astype(jnp.int32)
  den_a = _sc_sum_m0(msgs_0, idxc)
  den_b = _sc_sum_m1(msgs_1, idxc)
  w1t = W1.T
  dh0, dh1 = _tc_mlp(
      den_a, den_b,
      w1t[:F], w1t[F:], b1.reshape(1, F),
      W2.T, b2.reshape(1, F), L1W.T,
      S1.T, sb1.reshape(1, F), S2.T, sb2.reshape(1, F))
  return (dh0, dh1)
